# baseline (speedup 1.0000x reference)
import functools

import jax
import jax.numpy as jnp
from jax import lax
from jax.experimental import pallas as pl
from jax.experimental.pallas import tpu as pltpu
from jax.experimental.pallas import tpu_sc as plsc

B = 16384
D = 64
H1 = 8
_INFO = plsc.get_sparse_core_info()
NC = _INFO.num_cores
NS = _INFO.num_subcores
L = _INFO.num_lanes
NW = NC * NS
BPW = B // NW
NBLK = BPW // L


def _sc_body(uidx_h, iidx_h, ut_h, it_h, w1b_h, b1b_h, w2b_h, b2b_h,
             out_h,
             uidx_v, iidx_v, urows_v, irows_v, w1b_v, b1b_v, w2b_v, b2b_v,
             out_v, sem):
    wid = lax.axis_index("s") * NC + lax.axis_index("c")
    base = wid * BPW

    pltpu.sync_copy(uidx_h.at[pl.ds(base, BPW)], uidx_v)
    pltpu.sync_copy(iidx_h.at[pl.ds(base, BPW)], iidx_v)
    pltpu.sync_copy(w1b_h, w1b_v)
    pltpu.sync_copy(b1b_h, b1b_v)
    pltpu.sync_copy(w2b_h, w2b_v)
    pltpu.sync_copy(b2b_h, b2b_v)

    cu = pltpu.async_copy(ut_h.at[uidx_v], urows_v, sem)
    ci = pltpu.async_copy(it_h.at[iidx_v], irows_v, sem)
    cu.wait()
    ci.wait()

    def blk_body(blk, carry):
        rows = blk * L + lax.iota(jnp.int32, L)

        def d_body(d, accs):
            dsp = jnp.full((L,), d, dtype=jnp.int32)
            ucol = plsc.load_gather(urows_v, [rows, dsp])
            icol = plsc.load_gather(irows_v, [rows, dsp])
            x = ucol * icol
            return tuple(accs[j] + x * w1b_v[d, j] for j in range(H1))

        accs = lax.fori_loop(
            0, D, d_body,
            tuple(jnp.zeros((L,), jnp.float32) for _ in range(H1)),
            unroll=4)

        logit = b2b_v[...]
        for j in range(H1):
            h = jnp.maximum(accs[j] + b1b_v[j], 0.0)
            logit = logit + h * w2b_v[j]
        sig = 1.0 / (1.0 + jnp.exp(-logit))
        out_v[pl.ds(blk * L, L)] = sig
        return carry

    lax.fori_loop(0, NBLK, blk_body, 0)
    pltpu.sync_copy(out_v, out_h.at[pl.ds(base, BPW)])


@jax.jit
def _run(uidx, iidx, ut, it, w1b, b1b, w2b, b2b):
    mesh = plsc.VectorSubcoreMesh(core_axis_name="c", subcore_axis_name="s")
    f = pl.kernel(
        _sc_body,
        mesh=mesh,
        compiler_params=pltpu.CompilerParams(use_tc_tiling_on_sc=False,
                                             needs_layout_passes=False),
        out_type=jax.ShapeDtypeStruct((B,), jnp.float32),
        scratch_types=[
            pltpu.VMEM((BPW,), jnp.int32),
            pltpu.VMEM((BPW,), jnp.int32),
            pltpu.VMEM((BPW, D), jnp.float32),
            pltpu.VMEM((BPW, D), jnp.float32),
            pltpu.VMEM((D, H1, L), jnp.float32),
            pltpu.VMEM((H1, L), jnp.float32),
            pltpu.VMEM((H1, L), jnp.float32),
            pltpu.VMEM((L,), jnp.float32),
            pltpu.VMEM((BPW,), jnp.float32),
            pltpu.SemaphoreType.DMA,
        ],
    )
    return f(uidx, iidx, ut, it, w1b, b1b, w2b, b2b)


def kernel(group_inputs, user_inputs, item_inputs, user_table, item_table,
           W1, b1, W2, b2):
    del group_inputs
    uidx = user_inputs.astype(jnp.int32)
    iidx = item_inputs.astype(jnp.int32)
    w1b = jnp.broadcast_to(W1[:, :, None], (D, H1, L)).astype(jnp.float32)
    b1b = jnp.broadcast_to(b1[:, None], (H1, L)).astype(jnp.float32)
    w2b = jnp.broadcast_to(W2[:, 0][:, None], (H1, L)).astype(jnp.float32)
    b2b = jnp.broadcast_to(b2, (L,)).astype(jnp.float32)
    out = _run(uidx, iidx, user_table, item_table, w1b, b1b, w2b, b2b)
    return out.reshape(B, 1)
